# baseline (speedup 1.0000x reference)
import functools

import jax
import jax.numpy as jnp
from jax import lax
from jax.experimental import pallas as pl
from jax.experimental.pallas import tpu as pltpu
from jax.experimental.pallas import tpu_sc as plsc

NC = 2
NS = 16
L = 16


def _matmul(x, W):
    N, D = x.shape
    BM = 2000 if N % 2000 == 0 else N

    def body(x_ref, w_ref, o_ref):
        o_ref[...] = jnp.dot(x_ref[...], w_ref[...],
                             preferred_element_type=jnp.float32)

    return pl.pallas_call(
        body,
        grid=(N // BM,),
        in_specs=[
            pl.BlockSpec((BM, D), lambda i: (i, 0)),
            pl.BlockSpec((D, D), lambda i: (0, 0)),
        ],
        out_specs=pl.BlockSpec((BM, D), lambda i: (i, 0)),
        out_shape=jax.ShapeDtypeStruct((N, D), jnp.float32),
    )(x, W)


def _sc_gather_scatter(h, dst, src, w, N, E, D):
    NPASS = 4
    R = 8
    G = 32
    CHUNK = -(-N // (NC * NPASS * NS * L)) * NS * L
    RPT = CHUNK // NS
    EPT = E // NS
    B = max(b for b in range(L, min(2000, EPT) + 1, L) if EPT % b == 0)
    NB = EPT // B
    G = min(G, B)
    KB = -(-B // G)
    CAP = KB * G
    KBP = -(-CAP // 128)
    GPR = 128 // G
    NZF = RPT // G
    NZR = RPT - NZF * G
    NRB = RPT // L

    mesh = plsc.VectorSubcoreMesh(core_axis_name="c", subcore_axis_name="s")

    scratch = (
        [pltpu.VMEM((B,), jnp.int32),
         pltpu.VMEM((B,), jnp.int32),
         pltpu.VMEM((B,), jnp.float32)] * 2 +
        [pltpu.VMEM((KBP, 128), jnp.int32),
         pltpu.VMEM((KB, G), jnp.int32),
         pltpu.VMEM((CAP,), jnp.float32)] * 2 +
        [pltpu.VMEM((G, D), jnp.float32) for _ in range(R)] +
        [pltpu.VMEM_SHARED((CHUNK, D), jnp.float32)] +
        [pltpu.SMEM((1,), jnp.int32)] +
        [pltpu.SemaphoreType.DMA for _ in range(2 + 2 * R)]
    )

    @functools.partial(
        pl.kernel,
        out_type=jax.ShapeDtypeStruct((N, D), jnp.float32),
        mesh=mesh,
        scratch_types=scratch,
        compiler_params=pltpu.CompilerParams(needs_layout_passes=False),
    )
    def sc_kernel(h_hbm, dst_hbm, src_hbm, w_hbm, out_hbm, *scr):
        bufs0 = scr[0:3]
        bufs1 = scr[3:6]
        comp0 = scr[6:9]
        comp1 = scr[9:12]
        rbufs = scr[12:12 + R]
        acc = scr[12 + R]
        cnt_ref = scr[13 + R]
        esem0, esem1 = scr[14 + R:16 + R]
        gsems = scr[16 + R:16 + 2 * R]
        ssems = scr[16 + 2 * R:16 + 3 * R]

        cid = lax.axis_index("c")
        sid = lax.axis_index("s")
        zero16f = jnp.zeros((L,), jnp.float32)
        zero16i = jnp.zeros((L,), jnp.int32)
        rows0 = rbufs[0]

        def zc(i, _):
            for q in range(G // L):
                comp0[1][i, pl.ds(q * L, L)] = zero16i
                comp1[1][i, pl.ds(q * L, L)] = zero16i
            return 0
        lax.fori_loop(0, KB, zc, 0)

        def zs(i, _):
            for q in range(128 // L):
                comp0[0][i, pl.ds(q * L, L)] = zero16i
                comp1[0][i, pl.ds(q * L, L)] = zero16i
            return 0
        lax.fori_loop(0, KBP, zs, 0)

        def pass_body(p, _):
            lo = (NC * p + cid) * CHUNK
            hi = lo + CHUNK

            def zb(r, _):
                for k in range(D // L):
                    rows0[r, pl.ds(k * L, L)] = zero16f
                return 0
            lax.fori_loop(0, G, zb, 0)
            for q in range(NZF):
                pltpu.sync_copy(rows0, acc.at[pl.ds(sid * RPT + q * G, G)])
            if NZR:
                pltpu.sync_copy(rows0.at[pl.ds(0, NZR)],
                                acc.at[pl.ds(sid * RPT + NZF * G, NZR)])
            plsc.subcore_barrier()

            def eload(bi, bufs, esem):
                ebase = sid * EPT + bi * B
                pltpu.async_copy(dst_hbm.at[pl.ds(ebase, B)], bufs[0], esem)
                pltpu.async_copy(src_hbm.at[pl.ds(ebase, B)], bufs[1], esem)
                pltpu.async_copy(w_hbm.at[pl.ds(ebase, B)], bufs[2], esem)

            def ewait(bi, bufs, esem):
                ebase = sid * EPT + bi * B
                pltpu.make_async_copy(dst_hbm.at[pl.ds(ebase, B)], bufs[0],
                                      esem).wait()
                pltpu.make_async_copy(src_hbm.at[pl.ds(ebase, B)], bufs[1],
                                      esem).wait()
                pltpu.make_async_copy(w_hbm.at[pl.ds(ebase, B)], bufs[2],
                                      esem).wait()

            def compress(bufs, carrs):
                dstv, srcv, wv = bufs
                csrc, cdst, cw = carrs

                def comp(j, cnt_vec):
                    vd = dstv[pl.ds(j * L, L)]
                    vs = srcv[pl.ds(j * L, L)]
                    vw = wv[pl.ds(j * L, L)]
                    m = (vd >= lo) & (vd < hi)
                    mi = m.astype(jnp.int32)
                    pos = plsc.cumsum(mi) - mi + cnt_vec
                    prow = pos // G
                    pcol = pos - prow * G
                    srow = pos // 128
                    scol = pos - srow * 128
                    plsc.store_scatter(cdst, [prow, pcol], vd - lo, mask=m)
                    plsc.store_scatter(csrc, [srow, scol], vs, mask=m)
                    plsc.store_scatter(cw, [pos], vw, mask=m)
                    return cnt_vec + plsc.all_reduce_population_count(m)
                cnt_vec = lax.fori_loop(0, B // L, comp,
                                        jnp.zeros((L,), jnp.int32),
                                        unroll=2)
                cnt = jnp.max(cnt_vec, axis=0)

                for k in range(G // L):
                    ppos = lax.iota(jnp.int32, L) + (cnt + k * L)
                    plsc.store_scatter(cw, [ppos], zero16f, mask=ppos < CAP)
                return cnt

            def gidx(b, carrs):
                return carrs[0].at[b // GPR, pl.ds((b % GPR) * G, G)]

            def gfire(b, r, carrs):
                pltpu.async_copy(h_hbm.at[gidx(b, carrs)], rbufs[r],
                                 gsems[r])

            def gwait(b, r, carrs):
                pltpu.make_async_copy(h_hbm.at[gidx(b, carrs)], rbufs[r],
                                      gsems[r]).wait()

            def sfire(b, r, carrs):
                pltpu.async_copy(rbufs[r], acc.at[carrs[1].at[b]], ssems[r],
                                 add=True)

            def swait(b, r, carrs):
                pltpu.make_async_copy(rbufs[r], acc.at[carrs[1].at[b]],
                                      ssems[r]).wait()

            def batch_work(bi, cnt, bufs_n, esem_n, bufs_n2, esem_n2,
                           carrs, carrs_n):
                nb = (cnt + G - 1) // G

                for k in range(R):
                    @pl.when(k < nb)
                    def _(k=k):
                        gfire(k, k, carrs)

                @pl.when(bi + 1 < NB)
                def _():
                    ewait(bi + 1, bufs_n, esem_n)

                    @pl.when(bi + 2 < NB)
                    def _():
                        eload(bi + 2, bufs_n2, esem_n2)
                cnt_n = compress(bufs_n, carrs_n)

                def process(b, r):
                    pr = (r - 1) % R

                    @pl.when((b >= 1) & (b - 1 + R < nb))
                    def _():
                        swait(b - 1, pr, carrs)
                        gfire(b - 1 + R, pr, carrs)

                    gwait(b, r, carrs)
                    off = b * G
                    cw = carrs[2]

                    def mul(q, _):
                        wb = plsc.load_gather(
                            cw, [lax.broadcast(off + q, (L,))])
                        for k in range(D // L):
                            rbufs[r][q, pl.ds(k * L, L)] = (
                                rbufs[r][q, pl.ds(k * L, L)] * wb)
                        return 0
                    lax.fori_loop(0, G, mul, 0, unroll=2)

                    sfire(b, r, carrs)

                def chunk_body(b, _):
                    for r in range(R):
                        @pl.when(b % R == r)
                        def _(r=r):
                            process(b, r)
                    return 0
                lax.fori_loop(0, nb, chunk_body, 0)

                for k in range(R):
                    for r in range(R):
                        c = nb - R + k

                        @pl.when((c >= 0) & (c % R == r))
                        def _(c=c, r=r):
                            swait(c, r, carrs)
                cnt_ref[0] = cnt_n

            eload(0, bufs0, esem0)
            ewait(0, bufs0, esem0)
            if NB > 1:
                eload(1, bufs1, esem1)
            cnt0 = compress(bufs0, comp0)

            def batch_body(bi, cnt):
                @pl.when(bi % 2 == 0)
                def _():
                    batch_work(bi, cnt, bufs1, esem1, bufs0, esem0,
                               comp0, comp1)

                @pl.when(bi % 2 == 1)
                def _():
                    batch_work(bi, cnt, bufs0, esem0, bufs1, esem1,
                               comp1, comp0)
                return cnt_ref[0]
            lax.fori_loop(0, NB, batch_body, cnt0)
            plsc.subcore_barrier()

            row0 = lo + sid * RPT

            @pl.when(row0 + RPT <= N)
            def _():
                pltpu.sync_copy(acc.at[pl.ds(sid * RPT, RPT)],
                                out_hbm.at[pl.ds(row0, RPT)])

            @pl.when(row0 + RPT > N)
            def _():
                def cp(i, _):
                    g = row0 + i * L

                    @pl.when(g < N)
                    def _():
                        pltpu.sync_copy(acc.at[pl.ds(sid * RPT + i * L, L)],
                                        out_hbm.at[pl.ds(g, L)])
                    return 0
                lax.fori_loop(0, NRB, cp, 0)
            plsc.subcore_barrier()
            return 0
        lax.fori_loop(0, NPASS, pass_body, 0)

    return sc_kernel(h, dst, src, w)


def kernel(x, edge_index, edge_weight, W):
    N, D = x.shape
    E = edge_weight.shape[0]
    h = _matmul(x, W)
    return _sc_gather_scatter(h, edge_index[0], edge_index[1], edge_weight,
                              N=N, E=E, D=D)


# speedup vs baseline: 1.1134x; 1.1134x over previous
import functools

import jax
import jax.numpy as jnp
from jax import lax
from jax.experimental import pallas as pl
from jax.experimental.pallas import tpu as pltpu
from jax.experimental.pallas import tpu_sc as plsc

NC = 2
NS = 16
L = 16


def _matmul(x, W):
    N, D = x.shape
    BM = 2000 if N % 2000 == 0 else N

    def body(x_ref, w_ref, o_ref):
        o_ref[...] = jnp.dot(x_ref[...], w_ref[...],
                             preferred_element_type=jnp.float32)

    return pl.pallas_call(
        body,
        grid=(N // BM,),
        in_specs=[
            pl.BlockSpec((BM, D), lambda i: (i, 0)),
            pl.BlockSpec((D, D), lambda i: (0, 0)),
        ],
        out_specs=pl.BlockSpec((BM, D), lambda i: (i, 0)),
        out_shape=jax.ShapeDtypeStruct((N, D), jnp.float32),
    )(x, W)


def _sc_gather_scatter(h, dst, src, w, N, E, D):
    NPASS = 3
    R = 6
    G = 32
    CHUNK = -(-N // (NC * NPASS * NS * L)) * NS * L
    RPT = CHUNK // NS
    EPT = E // NS
    B = max(b for b in range(L, min(2000, EPT) + 1, L) if EPT % b == 0)
    NB = EPT // B
    G = min(G, B)
    KB = -(-B // G)
    CAP = KB * G
    KBP = -(-CAP // 128)
    GPR = 128 // G
    NZF = RPT // G
    NZR = RPT - NZF * G
    NRB = RPT // L

    mesh = plsc.VectorSubcoreMesh(core_axis_name="c", subcore_axis_name="s")

    scratch = (
        [pltpu.VMEM((B,), jnp.int32),
         pltpu.VMEM((B,), jnp.int32),
         pltpu.VMEM((B,), jnp.float32)] * 2 +
        [pltpu.VMEM((KBP, 128), jnp.int32),
         pltpu.VMEM((KB, G), jnp.int32),
         pltpu.VMEM((CAP,), jnp.float32)] * 2 +
        [pltpu.VMEM((G, D), jnp.float32) for _ in range(R)] +
        [pltpu.VMEM_SHARED((CHUNK, D), jnp.float32)] +
        [pltpu.SMEM((1,), jnp.int32)] +
        [pltpu.SemaphoreType.DMA for _ in range(2 + 2 * R)]
    )

    @functools.partial(
        pl.kernel,
        out_type=jax.ShapeDtypeStruct((N, D), jnp.float32),
        mesh=mesh,
        scratch_types=scratch,
        compiler_params=pltpu.CompilerParams(needs_layout_passes=False),
    )
    def sc_kernel(h_hbm, dst_hbm, src_hbm, w_hbm, out_hbm, *scr):
        bufs0 = scr[0:3]
        bufs1 = scr[3:6]
        comp0 = scr[6:9]
        comp1 = scr[9:12]
        rbufs = scr[12:12 + R]
        acc = scr[12 + R]
        cnt_ref = scr[13 + R]
        esem0, esem1 = scr[14 + R:16 + R]
        gsems = scr[16 + R:16 + 2 * R]
        ssems = scr[16 + 2 * R:16 + 3 * R]

        cid = lax.axis_index("c")
        sid = lax.axis_index("s")
        zero16f = jnp.zeros((L,), jnp.float32)
        zero16i = jnp.zeros((L,), jnp.int32)
        rows0 = rbufs[0]

        def zc(i, _):
            for q in range(G // L):
                comp0[1][i, pl.ds(q * L, L)] = zero16i
                comp1[1][i, pl.ds(q * L, L)] = zero16i
            return 0
        lax.fori_loop(0, KB, zc, 0)

        def zs(i, _):
            for q in range(128 // L):
                comp0[0][i, pl.ds(q * L, L)] = zero16i
                comp1[0][i, pl.ds(q * L, L)] = zero16i
            return 0
        lax.fori_loop(0, KBP, zs, 0)

        def pass_body(p, _):
            lo = (NC * p + cid) * CHUNK
            hi = lo + CHUNK

            def zb(r, _):
                for k in range(D // L):
                    rows0[r, pl.ds(k * L, L)] = zero16f
                return 0
            lax.fori_loop(0, G, zb, 0)
            for q in range(NZF):
                pltpu.sync_copy(rows0, acc.at[pl.ds(sid * RPT + q * G, G)])
            if NZR:
                pltpu.sync_copy(rows0.at[pl.ds(0, NZR)],
                                acc.at[pl.ds(sid * RPT + NZF * G, NZR)])
            plsc.subcore_barrier()

            def eload(bi, bufs, esem):
                ebase = sid * EPT + bi * B
                pltpu.async_copy(dst_hbm.at[pl.ds(ebase, B)], bufs[0], esem)
                pltpu.async_copy(src_hbm.at[pl.ds(ebase, B)], bufs[1], esem)
                pltpu.async_copy(w_hbm.at[pl.ds(ebase, B)], bufs[2], esem)

            def ewait(bi, bufs, esem):
                ebase = sid * EPT + bi * B
                pltpu.make_async_copy(dst_hbm.at[pl.ds(ebase, B)], bufs[0],
                                      esem).wait()
                pltpu.make_async_copy(src_hbm.at[pl.ds(ebase, B)], bufs[1],
                                      esem).wait()
                pltpu.make_async_copy(w_hbm.at[pl.ds(ebase, B)], bufs[2],
                                      esem).wait()

            def compress(bufs, carrs):
                dstv, srcv, wv = bufs
                csrc, cdst, cw = carrs

                def comp(j, cnt_vec):
                    vd = dstv[pl.ds(j * L, L)]
                    vs = srcv[pl.ds(j * L, L)]
                    vw = wv[pl.ds(j * L, L)]
                    m = (vd >= lo) & (vd < hi)
                    mi = m.astype(jnp.int32)
                    pos = plsc.cumsum(mi) - mi + cnt_vec
                    prow = pos // G
                    pcol = pos - prow * G
                    srow = pos // 128
                    scol = pos - srow * 128
                    plsc.store_scatter(cdst, [prow, pcol], vd - lo, mask=m)
                    plsc.store_scatter(csrc, [srow, scol], vs, mask=m)
                    plsc.store_scatter(cw, [pos], vw, mask=m)
                    return cnt_vec + plsc.all_reduce_population_count(m)
                cnt_vec = lax.fori_loop(0, B // L, comp,
                                        jnp.zeros((L,), jnp.int32),
                                        unroll=2)
                cnt = jnp.max(cnt_vec, axis=0)

                for k in range(G // L):
                    ppos = lax.iota(jnp.int32, L) + (cnt + k * L)
                    plsc.store_scatter(cw, [ppos], zero16f, mask=ppos < CAP)
                return cnt

            def gidx(b, carrs):
                return carrs[0].at[b // GPR, pl.ds((b % GPR) * G, G)]

            def gfire(b, r, carrs):
                pltpu.async_copy(h_hbm.at[gidx(b, carrs)], rbufs[r],
                                 gsems[r])

            def gwait(b, r, carrs):
                pltpu.make_async_copy(h_hbm.at[gidx(b, carrs)], rbufs[r],
                                      gsems[r]).wait()

            def sfire(b, r, carrs):
                pltpu.async_copy(rbufs[r], acc.at[carrs[1].at[b]], ssems[r],
                                 add=True)

            def swait(b, r, carrs):
                pltpu.make_async_copy(rbufs[r], acc.at[carrs[1].at[b]],
                                      ssems[r]).wait()

            def batch_work(bi, cnt, bufs_n, esem_n, bufs_n2, esem_n2,
                           carrs, carrs_n):
                nb = (cnt + G - 1) // G

                for k in range(R):
                    @pl.when(k < nb)
                    def _(k=k):
                        gfire(k, k, carrs)

                @pl.when(bi + 1 < NB)
                def _():
                    ewait(bi + 1, bufs_n, esem_n)

                    @pl.when(bi + 2 < NB)
                    def _():
                        eload(bi + 2, bufs_n2, esem_n2)
                cnt_n = compress(bufs_n, carrs_n)

                def process(b, r):
                    pr = (r - 1) % R

                    @pl.when((b >= 1) & (b - 1 + R < nb))
                    def _():
                        swait(b - 1, pr, carrs)
                        gfire(b - 1 + R, pr, carrs)

                    gwait(b, r, carrs)
                    off = b * G
                    cw = carrs[2]

                    def mul(q, _):
                        wb = plsc.load_gather(
                            cw, [lax.broadcast(off + q, (L,))])
                        for k in range(D // L):
                            rbufs[r][q, pl.ds(k * L, L)] = (
                                rbufs[r][q, pl.ds(k * L, L)] * wb)
                        return 0
                    lax.fori_loop(0, G, mul, 0, unroll=2)

                    sfire(b, r, carrs)

                def chunk_body(b, _):
                    for r in range(R):
                        @pl.when(b % R == r)
                        def _(r=r):
                            process(b, r)
                    return 0
                lax.fori_loop(0, nb, chunk_body, 0)

                for k in range(R):
                    for r in range(R):
                        c = nb - R + k

                        @pl.when((c >= 0) & (c % R == r))
                        def _(c=c, r=r):
                            swait(c, r, carrs)
                cnt_ref[0] = cnt_n

            eload(0, bufs0, esem0)
            ewait(0, bufs0, esem0)
            if NB > 1:
                eload(1, bufs1, esem1)
            cnt0 = compress(bufs0, comp0)

            def batch_body(bi, cnt):
                @pl.when(bi % 2 == 0)
                def _():
                    batch_work(bi, cnt, bufs1, esem1, bufs0, esem0,
                               comp0, comp1)

                @pl.when(bi % 2 == 1)
                def _():
                    batch_work(bi, cnt, bufs0, esem0, bufs1, esem1,
                               comp1, comp0)
                return cnt_ref[0]
            lax.fori_loop(0, NB, batch_body, cnt0)
            plsc.subcore_barrier()

            row0 = lo + sid * RPT

            @pl.when(row0 + RPT <= N)
            def _():
                pltpu.sync_copy(acc.at[pl.ds(sid * RPT, RPT)],
                                out_hbm.at[pl.ds(row0, RPT)])

            @pl.when(row0 + RPT > N)
            def _():
                def cp(i, _):
                    g = row0 + i * L

                    @pl.when(g < N)
                    def _():
                        pltpu.sync_copy(acc.at[pl.ds(sid * RPT + i * L, L)],
                                        out_hbm.at[pl.ds(g, L)])
                    return 0
                lax.fori_loop(0, NRB, cp, 0)
            plsc.subcore_barrier()
            return 0
        lax.fori_loop(0, NPASS, pass_body, 0)

    return sc_kernel(h, dst, src, w)


def kernel(x, edge_index, edge_weight, W):
    N, D = x.shape
    E = edge_weight.shape[0]
    h = _matmul(x, W)
    return _sc_gather_scatter(h, edge_index[0], edge_index[1], edge_weight,
                              N=N, E=E, D=D)
